# baseline (speedup 1.0000x reference)
import functools

import numpy as np

import jax
import jax.numpy as jnp
from jax import lax
from jax.experimental import pallas as pl
from jax.experimental.pallas import tpu as pltpu
from jax.experimental.pallas import tpu_sc as plsc

N = 10000
E = 160000
NF = 256
NH = 512

NP = 10240
EP = 163840
ER = EP // 128
RPT = ER // 16
NSL = NP // 16
EB = 64
NBUF = 4
ERW = EP // EB
SPT = ERW // 16
CHUNK = 32

_F32 = jnp.float32
_I32 = jnp.int32

_PERM128 = np.concatenate(
    [np.concatenate([32 * q + 2 * np.arange(16),
                     32 * q + 2 * np.arange(16) + 1]) for q in range(4)])
_PERMF = np.concatenate([_PERM128, 128 + _PERM128])


def _fast_rsqrt(d):
    ih = plsc.bitcast(d, _I32)
    ih = jnp.int32(0x5F3759DF) - lax.shift_right_logical(ih, 1)
    y = plsc.bitcast(ih, _F32)
    for _ in range(3):
        y = y * (1.5 - 0.5 * d * y * y)
    return y


def _k1_body(x_hbm, dst_hbm, dinv_hbm, xp0_hbm, xp1_hbm,
             idx_v, deg_v, dv_v, tmp_v, xb_v, shd):
    c = lax.axis_index("c")
    s = lax.axis_index("s")
    base = s * NSL

    pltpu.sync_copy(dst_hbm.at[pl.ds(s * SPT, SPT)], idx_v)

    zeros16 = jnp.zeros((16,), _F32)

    def _zero(i, _):
        deg_v[pl.ds(pl.multiple_of(i * 16, 16), 16)] = zeros16
        return 0

    lax.fori_loop(0, NP // 16, _zero, 0)

    ones16 = jnp.ones((16,), _F32)

    def _scat(j, _):
        for k in range(EB // 16):
            iv = idx_v[j, pl.ds(k * 16, 16)]
            plsc.addupdate_scatter(deg_v, [iv], ones16)
        return 0

    lax.fori_loop(0, SPT, _scat, 0)

    pltpu.sync_copy(deg_v, shd.at[s])
    plsc.subcore_barrier()

    pltpu.sync_copy(shd.at[pl.ds(0, 16), pl.ds(base, NSL)], tmp_v)

    def _acc(i, _):
        sl = pl.ds(pl.multiple_of(i * 16, 16), 16)
        acc = tmp_v[0, sl]
        for t in range(1, 16):
            acc = acc + tmp_v[t, sl]
        dv_v[sl] = acc
        return 0

    lax.fori_loop(0, NSL // 16, _acc, 0)

    def _rs(i, _):
        sl = pl.ds(pl.multiple_of(i * 16, 16), 16)
        dv_v[sl] = _fast_rsqrt(dv_v[sl] + 1.0)
        return 0

    lax.fori_loop(0, NSL // 16, _rs, 0)

    @pl.when(c == 0)
    def _():
        pltpu.sync_copy(dv_v, dinv_hbm.at[pl.ds(base, NSL)])

    half = NSL // 2
    for h in range(2):
        r0 = base + h * half
        pltpu.sync_copy(
            x_hbm.at[pl.ds(r0, half), pl.ds(pl.multiple_of(c * 128, 128), 128)],
            xb_v)

        def _scale(i, _):
            ridx = jnp.full((16,), h * half + i, _I32)
            dsp = plsc.load_gather(dv_v, [ridx])
            for k in range(8):
                sl = pl.ds(k * 16, 16)
                xb_v[i, sl] = xb_v[i, sl] * dsp
            return 0

        lax.fori_loop(0, half, _scale, 0)

        @pl.when(c == 0)
        def _():
            pltpu.sync_copy(xb_v, xp0_hbm.at[pl.ds(r0, half)])

        @pl.when(c == 1)
        def _():
            pltpu.sync_copy(xb_v, xp1_hbm.at[pl.ds(r0, half)])


def _k2_body(xp0_hbm, xp1_hbm, src_hbm, dst_hbm, agg0_hbm, agg1_hbm, *scr):
    c = lax.axis_index("c")
    s = lax.axis_index("s")
    sidxs = scr[0:2]
    didxs = scr[2:4]
    gbufs = scr[4:4 + NBUF]
    sbufs = scr[4 + NBUF:6 + NBUF]
    acc_sh = scr[6 + NBUF]
    gsems = scr[7 + NBUF:7 + 2 * NBUF]
    ssems = scr[7 + 2 * NBUF:9 + 2 * NBUF]
    isems = scr[9 + 2 * NBUF:11 + 2 * NBUF]

    zeros16 = jnp.zeros((16,), _F32)

    def _zero(i, _):
        for k in range(8):
            sbufs[0][i, pl.ds(k * 16, 16)] = zeros16
        return 0

    lax.fori_loop(0, EB, _zero, 0)

    for h in range(NSL // EB):
        pltpu.sync_copy(sbufs[0], acc_sh.at[pl.ds(s * NSL + h * EB, EB)])
    plsc.subcore_barrier()

    def _start_gather(j, b, iv):
        @pl.when(c == 0)
        def _():
            pltpu.async_copy(xp0_hbm.at[iv.at[j]], gbufs[b], gsems[b])

        @pl.when(c == 1)
        def _():
            pltpu.async_copy(xp1_hbm.at[iv.at[j]], gbufs[b], gsems[b])

    def _wait_gather(j, b, iv):
        @pl.when(c == 0)
        def _():
            pltpu.make_async_copy(xp0_hbm.at[iv.at[j]], gbufs[b],
                                  gsems[b]).wait()

        @pl.when(c == 1)
        def _():
            pltpu.make_async_copy(xp1_hbm.at[iv.at[j]], gbufs[b],
                                  gsems[b]).wait()

    def _start_scatter(j, b, iv):
        pltpu.async_copy(sbufs[b], acc_sh.at[iv.at[j]], ssems[b], add=True)

    def _wait_scatter(j, b, iv):
        pltpu.make_async_copy(sbufs[b], acc_sh.at[iv.at[j]], ssems[b]).wait()

    hi_mask = jnp.int32(-65536)

    def _unpack(gb, sb):
        def _row(i, _):
            for q in range(4):
                v = gbufs[gb][i, pl.ds(q * 16, 16)]
                lo = plsc.bitcast(lax.shift_left(v, 16), _F32)
                hi = plsc.bitcast(jnp.bitwise_and(v, hi_mask), _F32)
                sbufs[sb][i, pl.ds(q * 32, 16)] = lo
                sbufs[sb][i, pl.ds(q * 32 + 16, 16)] = hi
            return 0

        lax.fori_loop(0, EB, _row, 0)

    def _stage_idx(ck, sync):
        row0 = s * SPT + ck * CHUNK
        pq = ck % 2
        if sync:
            pltpu.sync_copy(src_hbm.at[pl.ds(row0, CHUNK)], sidxs[pq])
            pltpu.sync_copy(dst_hbm.at[pl.ds(row0, CHUNK)], didxs[pq])
        else:
            pltpu.async_copy(src_hbm.at[pl.ds(row0, CHUNK)], sidxs[pq],
                             isems[0])
            pltpu.async_copy(dst_hbm.at[pl.ds(row0, CHUNK)], didxs[pq],
                             isems[1])

    def _wait_idx(ck):
        row0 = s * SPT + ck * CHUNK
        pq = ck % 2
        pltpu.make_async_copy(src_hbm.at[pl.ds(row0, CHUNK)], sidxs[pq],
                              isems[0]).wait()
        pltpu.make_async_copy(dst_hbm.at[pl.ds(row0, CHUNK)], didxs[pq],
                              isems[1]).wait()

    NCK = SPT // CHUNK
    _stage_idx(0, True)
    _start_gather(0, 0, sidxs[0])
    _start_gather(1, 1, sidxs[0])
    _start_gather(2, 2, sidxs[0])

    for ck in range(NCK):
        si = sidxs[ck % 2]
        di = didxs[ck % 2]
        dip = didxs[1 - ck % 2]
        _wait_gather(0, 0, si)
        if ck > 0:
            _wait_scatter(CHUNK - 2, 0, dip)
        _unpack(0, 0)
        _start_scatter(0, 0, di)
        _start_gather(3, 3, si)
        _wait_gather(1, 1, si)
        if ck > 0:
            _wait_scatter(CHUNK - 1, 1, dip)
        if ck < NCK - 1:
            _stage_idx(ck + 1, False)
        _unpack(1, 1)
        _start_scatter(1, 1, di)
        _start_gather(4, 0, si)

        @pl.loop(2, CHUNK - 6, step=4)
        def _edge(g):
            for db in range(4):
                j = g + db
                gb = (2 + db) % 4
                sb = db % 2
                _wait_gather(j, gb, si)
                _wait_scatter(j - 2, sb, di)
                _unpack(gb, sb)
                _start_scatter(j, sb, di)
                _start_gather(j + 3, (1 + db) % 4, si)

        for jj in range(CHUNK - 6, CHUNK):
            _wait_gather(jj, jj % 4, si)
            _wait_scatter(jj - 2, jj % 2, di)
            _unpack(jj % 4, jj % 2)
            _start_scatter(jj, jj % 2, di)
            if jj + 3 <= CHUNK - 1:
                _start_gather(jj + 3, (jj + 3) % 4, si)
            elif ck < NCK - 1:
                if jj == CHUNK - 3:
                    _wait_idx(ck + 1)
                nsi = sidxs[(ck + 1) % 2]
                _start_gather(jj - (CHUNK - 3), (jj - (CHUNK - 3)) % 4, nsi)
    _wait_scatter(CHUNK - 2, 0, didxs[(NCK - 1) % 2])
    _wait_scatter(CHUNK - 1, 1, didxs[(NCK - 1) % 2])

    plsc.subcore_barrier()

    for h in range(NSL // EB):
        rows = pl.ds(s * NSL + h * EB, EB)
        pltpu.sync_copy(acc_sh.at[rows], sbufs[0])

        @pl.when(c == 0)
        def _():
            pltpu.sync_copy(sbufs[0], agg0_hbm.at[rows])

        @pl.when(c == 1)
        def _():
            pltpu.sync_copy(sbufs[0], agg1_hbm.at[rows])


def _k3_body(agg0, agg1, xp0, xp1, dinv, w, b2, a2, u2, out_ref):
    w_ = w[...]
    u_ = u2[...]
    wv = jnp.dot(u_, w_, preferred_element_type=_F32)
    nv = jnp.sqrt(jnp.sum(wv * wv))
    v = wv / (nv + 1e-12)
    wv2 = lax.dot_general(v, w_, (((1,), (1,)), ((), ())),
                          preferred_element_type=_F32)
    nu = jnp.sqrt(jnp.sum(wv2 * wv2))
    sigma = jnp.sum(wv2 * wv2) / (nu + 1e-12)

    d = dinv[...]
    t0 = (agg0[...] + xp0[...]) * d
    t1 = (agg1[...] + xp1[...]) * d
    o = (jnp.dot(t0, w_[0:128, :], preferred_element_type=_F32)
         + jnp.dot(t1, w_[128:256, :], preferred_element_type=_F32))
    o = o * (1.0 / sigma) + b2[...]
    al = a2[0, 0]
    out_ref[...] = jnp.where(o >= 0, o, al * o)


@jax.jit
def kernel(x, edge_index, W, b, a, u):
    src = edge_index[0]
    dst = edge_index[1]
    pad = jnp.full((EP - E,), N, _I32)
    srcr = jnp.concatenate([src, pad]).reshape(ERW, EB)
    dstr = jnp.concatenate([dst, pad]).reshape(ERW, EB)
    x_pad = jnp.pad(x, ((0, NP - N), (0, 0)))

    mesh = plsc.VectorSubcoreMesh(core_axis_name="c", subcore_axis_name="s")

    k1 = pl.kernel(
        _k1_body,
        out_type=(
            jax.ShapeDtypeStruct((NP,), _F32),
            jax.ShapeDtypeStruct((NP, 128), _F32),
            jax.ShapeDtypeStruct((NP, 128), _F32),
        ),
        mesh=mesh,
        scratch_types=[
            pltpu.VMEM((SPT, EB), _I32),
            pltpu.VMEM((NP,), _F32),
            pltpu.VMEM((NSL,), _F32),
            pltpu.VMEM((16, NSL), _F32),
            pltpu.VMEM((NSL // 2, 128), _F32),
            pltpu.VMEM_SHARED((16, NP), _F32),
        ],
        compiler_params=pltpu.CompilerParams(needs_layout_passes=False),
    )
    dinv, xp0, xp1 = k1(x_pad, dstr)

    k2 = pl.kernel(
        _k2_body,
        out_type=(
            jax.ShapeDtypeStruct((NP, 128), _F32),
            jax.ShapeDtypeStruct((NP, 128), _F32),
        ),
        mesh=mesh,
        scratch_types=[pltpu.VMEM((CHUNK, EB), _I32)] * 4
        + [pltpu.VMEM((EB, 64), _I32)] * NBUF
        + [pltpu.VMEM((EB, 128), _F32)] * 2
        + [pltpu.VMEM_SHARED((NP, 128), _F32)]
        + [pltpu.SemaphoreType.DMA] * (2 * NBUF + 4),
        compiler_params=pltpu.CompilerParams(needs_layout_passes=False,
                                             use_tc_tiling_on_sc=False),
    )
    xp0i = lax.bitcast_convert_type(
        xp0.astype(jnp.bfloat16).reshape(NP, 64, 2), _I32)
    xp1i = lax.bitcast_convert_type(
        xp1.astype(jnp.bfloat16).reshape(NP, 64, 2), _I32)
    agg0, agg1 = k2(xp0i, xp1i, srcr, dstr)
    xp0p = jnp.take(xp0, _PERM128, axis=1)
    xp1p = jnp.take(xp1, _PERM128, axis=1)
    w_p = W[_PERMF, :]
    u_p = u[_PERMF]

    blk = 512
    grid = NP // blk
    outp = pl.pallas_call(
        _k3_body,
        grid=(grid,),
        in_specs=[
            pl.BlockSpec((blk, 128), lambda i: (i, 0)),
            pl.BlockSpec((blk, 128), lambda i: (i, 0)),
            pl.BlockSpec((blk, 128), lambda i: (i, 0)),
            pl.BlockSpec((blk, 128), lambda i: (i, 0)),
            pl.BlockSpec((blk, 1), lambda i: (i, 0)),
            pl.BlockSpec((NF, NH), lambda i: (0, 0)),
            pl.BlockSpec((1, NH), lambda i: (0, 0)),
            pl.BlockSpec((1, 1), lambda i: (0, 0)),
            pl.BlockSpec((1, NF), lambda i: (0, 0)),
        ],
        out_specs=pl.BlockSpec((blk, NH), lambda i: (i, 0)),
        out_shape=jax.ShapeDtypeStruct((NP, NH), _F32),
    )(agg0, agg1, xp0p, xp1p, dinv.reshape(NP, 1), w_p,
      b.reshape(1, NH), a.reshape(1, 1), u_p.reshape(1, NF))

    return outp[:N]


# speedup vs baseline: 1.2978x; 1.2978x over previous
import functools

import jax
import jax.numpy as jnp
from jax import lax
from jax.experimental import pallas as pl
from jax.experimental.pallas import tpu as pltpu
from jax.experimental.pallas import tpu_sc as plsc

N = 10000
E = 160000
NF = 256
NH = 512

NP = 10240
EP = 163840
ER = EP // 128
RPT = ER // 16
NSL = NP // 16
EB = 64
NBUF = 4
ERW = EP // EB
SPT = ERW // 16
CHUNK = 32

_F32 = jnp.float32
_I32 = jnp.int32


def _fast_rsqrt(d):
    ih = plsc.bitcast(d, _I32)
    ih = jnp.int32(0x5F3759DF) - lax.shift_right_logical(ih, 1)
    y = plsc.bitcast(ih, _F32)
    for _ in range(3):
        y = y * (1.5 - 0.5 * d * y * y)
    return y


def _k1_body(x_hbm, dst_hbm, dinv_hbm, xp0_hbm, xp1_hbm,
             idx_v, deg_v, dv_v, tmp_v, xb_v, shd):
    c = lax.axis_index("c")
    s = lax.axis_index("s")
    base = s * NSL

    pltpu.sync_copy(dst_hbm.at[pl.ds(s * SPT, SPT)], idx_v)

    zeros16 = jnp.zeros((16,), _F32)

    def _zero(i, _):
        deg_v[pl.ds(pl.multiple_of(i * 16, 16), 16)] = zeros16
        return 0

    lax.fori_loop(0, NP // 16, _zero, 0)

    ones16 = jnp.ones((16,), _F32)

    def _scat(j, _):
        for k in range(EB // 16):
            iv = idx_v[j, pl.ds(k * 16, 16)]
            plsc.addupdate_scatter(deg_v, [iv], ones16)
        return 0

    lax.fori_loop(0, SPT, _scat, 0)

    pltpu.sync_copy(deg_v, shd.at[s])
    plsc.subcore_barrier()

    pltpu.sync_copy(shd.at[pl.ds(0, 16), pl.ds(base, NSL)], tmp_v)

    def _acc(i, _):
        sl = pl.ds(pl.multiple_of(i * 16, 16), 16)
        acc = tmp_v[0, sl]
        for t in range(1, 16):
            acc = acc + tmp_v[t, sl]
        dv_v[sl] = acc
        return 0

    lax.fori_loop(0, NSL // 16, _acc, 0)

    def _rs(i, _):
        sl = pl.ds(pl.multiple_of(i * 16, 16), 16)
        dv_v[sl] = _fast_rsqrt(dv_v[sl] + 1.0)
        return 0

    lax.fori_loop(0, NSL // 16, _rs, 0)

    @pl.when(c == 0)
    def _():
        pltpu.sync_copy(dv_v, dinv_hbm.at[pl.ds(base, NSL)])

    half = NSL // 2
    for h in range(2):
        r0 = base + h * half
        pltpu.sync_copy(
            x_hbm.at[pl.ds(r0, half), pl.ds(pl.multiple_of(c * 128, 128), 128)],
            xb_v)

        def _scale(i, _):
            ridx = jnp.full((16,), h * half + i, _I32)
            dsp = plsc.load_gather(dv_v, [ridx])
            for k in range(8):
                sl = pl.ds(k * 16, 16)
                xb_v[i, sl] = xb_v[i, sl] * dsp
            return 0

        lax.fori_loop(0, half, _scale, 0)

        @pl.when(c == 0)
        def _():
            pltpu.sync_copy(xb_v, xp0_hbm.at[pl.ds(r0, half)])

        @pl.when(c == 1)
        def _():
            pltpu.sync_copy(xb_v, xp1_hbm.at[pl.ds(r0, half)])


def _k2_body(xp0_hbm, xp1_hbm, src_hbm, dst_hbm, agg0_hbm, agg1_hbm, *scr):
    c = lax.axis_index("c")
    s = lax.axis_index("s")
    sidxs = scr[0:2]
    didxs = scr[2:4]
    bufs = scr[4:4 + NBUF]
    acc_sh = scr[4 + NBUF]
    gsems = scr[5 + NBUF:5 + 2 * NBUF]
    ssems = scr[5 + 2 * NBUF:5 + 3 * NBUF]
    isems = scr[5 + 3 * NBUF:7 + 3 * NBUF]

    zeros16 = jnp.zeros((16,), _F32)

    def _zero(i, _):
        for k in range(8):
            bufs[0][i, pl.ds(k * 16, 16)] = zeros16
        return 0

    lax.fori_loop(0, EB, _zero, 0)

    for h in range(NSL // EB):
        pltpu.sync_copy(bufs[0], acc_sh.at[pl.ds(s * NSL + h * EB, EB)])
    plsc.subcore_barrier()

    def _start_gather(j, b, iv):
        @pl.when(c == 0)
        def _():
            pltpu.async_copy(xp0_hbm.at[iv.at[j]], bufs[b], gsems[b])

        @pl.when(c == 1)
        def _():
            pltpu.async_copy(xp1_hbm.at[iv.at[j]], bufs[b], gsems[b])

    def _wait_gather(j, b, iv):
        @pl.when(c == 0)
        def _():
            pltpu.make_async_copy(xp0_hbm.at[iv.at[j]], bufs[b],
                                  gsems[b]).wait()

        @pl.when(c == 1)
        def _():
            pltpu.make_async_copy(xp1_hbm.at[iv.at[j]], bufs[b],
                                  gsems[b]).wait()

    def _start_scatter(j, b, iv):
        pltpu.async_copy(bufs[b], acc_sh.at[iv.at[j]], ssems[b], add=True)

    def _wait_scatter(j, b, iv):
        pltpu.make_async_copy(bufs[b], acc_sh.at[iv.at[j]], ssems[b]).wait()

    def _stage_idx(ck, sync):
        row0 = s * SPT + ck * CHUNK
        p = ck % 2
        if sync:
            pltpu.sync_copy(src_hbm.at[pl.ds(row0, CHUNK)], sidxs[p])
            pltpu.sync_copy(dst_hbm.at[pl.ds(row0, CHUNK)], didxs[p])
        else:
            pltpu.async_copy(src_hbm.at[pl.ds(row0, CHUNK)], sidxs[p],
                             isems[0])
            pltpu.async_copy(dst_hbm.at[pl.ds(row0, CHUNK)], didxs[p],
                             isems[1])

    def _wait_idx(ck):
        row0 = s * SPT + ck * CHUNK
        p = ck % 2
        pltpu.make_async_copy(src_hbm.at[pl.ds(row0, CHUNK)], sidxs[p],
                              isems[0]).wait()
        pltpu.make_async_copy(dst_hbm.at[pl.ds(row0, CHUNK)], didxs[p],
                              isems[1]).wait()

    NCK = SPT // CHUNK
    _stage_idx(0, True)
    for q in range(NBUF - 1):
        _start_gather(q, q, sidxs[0])

    for ck in range(NCK):
        si = sidxs[ck % 2]
        di = didxs[ck % 2]
        _wait_gather(0, 0, si)
        _start_scatter(0, 0, di)
        if ck > 0:
            _wait_scatter(CHUNK - 1, NBUF - 1, didxs[1 - ck % 2])
        if ck < NCK - 1:
            _stage_idx(ck + 1, False)
        _start_gather(NBUF - 1, NBUF - 1, si)

        @pl.loop(1, CHUNK - NBUF + 1, step=NBUF)
        def _edge(g):
            for db in range(NBUF):
                j = g + db
                b = (1 + db) % NBUF
                _wait_gather(j, b, si)
                _start_scatter(j, b, di)
                _wait_scatter(j - 1, db % NBUF, di)
                _start_gather(j + NBUF - 1, db % NBUF, si)

        for jj in range(CHUNK - NBUF + 1, CHUNK):
            _wait_gather(jj, jj % NBUF, si)
            _start_scatter(jj, jj % NBUF, di)
            _wait_scatter(jj - 1, (jj - 1) % NBUF, di)
        if ck < NCK - 1:
            _wait_idx(ck + 1)
            nsi = sidxs[(ck + 1) % 2]
            for q in range(NBUF - 1):
                _start_gather(q, q, nsi)
    _wait_scatter(CHUNK - 1, NBUF - 1, didxs[(NCK - 1) % 2])

    plsc.subcore_barrier()

    for h in range(NSL // EB):
        rows = pl.ds(s * NSL + h * EB, EB)
        pltpu.sync_copy(acc_sh.at[rows], bufs[0])

        @pl.when(c == 0)
        def _():
            pltpu.sync_copy(bufs[0], agg0_hbm.at[rows])

        @pl.when(c == 1)
        def _():
            pltpu.sync_copy(bufs[0], agg1_hbm.at[rows])


def _k3_body(agg0, agg1, xp0, xp1, dinv, w, b2, a2, u2, out_ref):
    w_ = w[...]
    u_ = u2[...]
    wv = jnp.dot(u_, w_, preferred_element_type=_F32)
    nv = jnp.sqrt(jnp.sum(wv * wv))
    v = wv / (nv + 1e-12)
    wv2 = lax.dot_general(v, w_, (((1,), (1,)), ((), ())),
                          preferred_element_type=_F32)
    nu = jnp.sqrt(jnp.sum(wv2 * wv2))
    sigma = jnp.sum(wv2 * wv2) / (nu + 1e-12)

    d = dinv[...]
    t0 = (agg0[...] + xp0[...]) * d
    t1 = (agg1[...] + xp1[...]) * d
    o = (jnp.dot(t0, w_[0:128, :], preferred_element_type=_F32)
         + jnp.dot(t1, w_[128:256, :], preferred_element_type=_F32))
    o = o * (1.0 / sigma) + b2[...]
    al = a2[0, 0]
    out_ref[...] = jnp.where(o >= 0, o, al * o)


@jax.jit
def kernel(x, edge_index, W, b, a, u):
    src = edge_index[0]
    dst = edge_index[1]
    pad = jnp.full((EP - E,), N, _I32)
    srcr = jnp.concatenate([src, pad]).reshape(ERW, EB)
    dstr = jnp.concatenate([dst, pad]).reshape(ERW, EB)
    x_pad = jnp.pad(x, ((0, NP - N), (0, 0)))

    mesh = plsc.VectorSubcoreMesh(core_axis_name="c", subcore_axis_name="s")

    k1 = pl.kernel(
        _k1_body,
        out_type=(
            jax.ShapeDtypeStruct((NP,), _F32),
            jax.ShapeDtypeStruct((NP, 128), _F32),
            jax.ShapeDtypeStruct((NP, 128), _F32),
        ),
        mesh=mesh,
        scratch_types=[
            pltpu.VMEM((SPT, EB), _I32),
            pltpu.VMEM((NP,), _F32),
            pltpu.VMEM((NSL,), _F32),
            pltpu.VMEM((16, NSL), _F32),
            pltpu.VMEM((NSL // 2, 128), _F32),
            pltpu.VMEM_SHARED((16, NP), _F32),
        ],
        compiler_params=pltpu.CompilerParams(needs_layout_passes=False),
    )
    dinv, xp0, xp1 = k1(x_pad, dstr)

    k2 = pl.kernel(
        _k2_body,
        out_type=(
            jax.ShapeDtypeStruct((NP, 128), _F32),
            jax.ShapeDtypeStruct((NP, 128), _F32),
        ),
        mesh=mesh,
        scratch_types=[pltpu.VMEM((CHUNK, EB), _I32)] * 4
        + [pltpu.VMEM((EB, 128), _F32)] * NBUF
        + [pltpu.VMEM_SHARED((NP, 128), _F32)]
        + [pltpu.SemaphoreType.DMA] * (2 * NBUF + 2),
        compiler_params=pltpu.CompilerParams(needs_layout_passes=False),
    )
    agg0, agg1 = k2(xp0, xp1, srcr, dstr)

    blk = 512
    grid = NP // blk
    outp = pl.pallas_call(
        _k3_body,
        grid=(grid,),
        in_specs=[
            pl.BlockSpec((blk, 128), lambda i: (i, 0)),
            pl.BlockSpec((blk, 128), lambda i: (i, 0)),
            pl.BlockSpec((blk, 128), lambda i: (i, 0)),
            pl.BlockSpec((blk, 128), lambda i: (i, 0)),
            pl.BlockSpec((blk, 1), lambda i: (i, 0)),
            pl.BlockSpec((NF, NH), lambda i: (0, 0)),
            pl.BlockSpec((1, NH), lambda i: (0, 0)),
            pl.BlockSpec((1, 1), lambda i: (0, 0)),
            pl.BlockSpec((1, NF), lambda i: (0, 0)),
        ],
        out_specs=pl.BlockSpec((blk, NH), lambda i: (i, 0)),
        out_shape=jax.ShapeDtypeStruct((N, NH), _F32),
    )(agg0, agg1, xp0, xp1, dinv.reshape(NP, 1), W,
      b.reshape(1, NH), a.reshape(1, 1), u.reshape(1, NF))

    return outp


# speedup vs baseline: 1.3233x; 1.0197x over previous
import functools

import jax
import jax.numpy as jnp
from jax import lax
from jax.experimental import pallas as pl
from jax.experimental.pallas import tpu as pltpu
from jax.experimental.pallas import tpu_sc as plsc

N = 10000
E = 160000
NF = 256
NH = 512

NP = 10240
EP = 163840
ER = EP // 128
RPT = ER // 16
NSL = NP // 16
EB = 64
NBUF = 4
ERW = EP // EB
SPT = ERW // 16
CHUNK = 32

_F32 = jnp.float32
_I32 = jnp.int32


def _fast_rsqrt(d):
    ih = plsc.bitcast(d, _I32)
    ih = jnp.int32(0x5F3759DF) - lax.shift_right_logical(ih, 1)
    y = plsc.bitcast(ih, _F32)
    for _ in range(3):
        y = y * (1.5 - 0.5 * d * y * y)
    return y


def _k1_body(x_hbm, dst_hbm, dinv_hbm, xp0_hbm, xp1_hbm,
             idx_v, deg_v, dv_v, tmp_v, xb_v, xb2_v, shd, xisem, xosem):
    c = lax.axis_index("c")
    s = lax.axis_index("s")
    base = s * NSL

    pltpu.sync_copy(dst_hbm.at[pl.ds(s * SPT, SPT)], idx_v)

    zeros16 = jnp.zeros((16,), _F32)

    def _zero(i, _):
        deg_v[pl.ds(pl.multiple_of(i * 16, 16), 16)] = zeros16
        return 0

    lax.fori_loop(0, NP // 16, _zero, 0)

    ones16 = jnp.ones((16,), _F32)

    def _scat(j, _):
        for k in range(EB // 16):
            iv = idx_v[j, pl.ds(k * 16, 16)]
            plsc.addupdate_scatter(deg_v, [iv], ones16)
        return 0

    lax.fori_loop(0, SPT, _scat, 0)

    pltpu.sync_copy(deg_v, shd.at[s])
    plsc.subcore_barrier()

    pltpu.sync_copy(shd.at[pl.ds(0, 16), pl.ds(base, NSL)], tmp_v)

    def _acc(i, _):
        sl = pl.ds(pl.multiple_of(i * 16, 16), 16)
        acc = tmp_v[0, sl]
        for t in range(1, 16):
            acc = acc + tmp_v[t, sl]
        dv_v[sl] = acc
        return 0

    lax.fori_loop(0, NSL // 16, _acc, 0)

    def _rs(i, _):
        sl = pl.ds(pl.multiple_of(i * 16, 16), 16)
        dv_v[sl] = _fast_rsqrt(dv_v[sl] + 1.0)
        return 0

    lax.fori_loop(0, NSL // 16, _rs, 0)

    @pl.when(c == 0)
    def _():
        pltpu.sync_copy(dv_v, dinv_hbm.at[pl.ds(base, NSL)])

    qr = NSL // 4
    xbs = (xb_v, xb2_v)

    def _xin(h):
        return pltpu.make_async_copy(
            x_hbm.at[pl.ds(base + h * qr, qr),
                     pl.ds(pl.multiple_of(c * 128, 128), 128)],
            xbs[h % 2], xisem)

    def _xout(h):
        def _mk0():
            return pltpu.make_async_copy(
                xbs[h % 2], xp0_hbm.at[pl.ds(base + h * qr, qr)], xosem)

        def _mk1():
            return pltpu.make_async_copy(
                xbs[h % 2], xp1_hbm.at[pl.ds(base + h * qr, qr)], xosem)

        return _mk0, _mk1

    _xin(0).start()
    _xin(1).start()
    for h in range(4):
        _xin(h).wait()

        def _scale(i, _):
            ridx = jnp.full((16,), h * qr + i, _I32)
            dsp = plsc.load_gather(dv_v, [ridx])
            for k in range(8):
                sl = pl.ds(k * 16, 16)
                xbs[h % 2][i, sl] = xbs[h % 2][i, sl] * dsp
            return 0

        lax.fori_loop(0, qr, _scale, 0)

        mk0, mk1 = _xout(h)

        @pl.when(c == 0)
        def _():
            mk0().start()

        @pl.when(c == 1)
        def _():
            mk1().start()

        if h + 2 < 4:
            @pl.when(c == 0)
            def _():
                mk0().wait()

            @pl.when(c == 1)
            def _():
                mk1().wait()

            _xin(h + 2).start()
    for h in range(2, 4):
        mk0, mk1 = _xout(h)

        @pl.when(c == 0)
        def _():
            mk0().wait()

        @pl.when(c == 1)
        def _():
            mk1().wait()


def _k2_body(xp0_hbm, xp1_hbm, src_hbm, dst_hbm, agg0_hbm, agg1_hbm, *scr):
    c = lax.axis_index("c")
    s = lax.axis_index("s")
    sidxs = scr[0:2]
    didxs = scr[2:4]
    bufs = scr[4:4 + NBUF]
    acc_sh = scr[4 + NBUF]
    gsems = scr[5 + NBUF:5 + 2 * NBUF]
    ssems = scr[5 + 2 * NBUF:5 + 3 * NBUF]
    isems = scr[5 + 3 * NBUF:7 + 3 * NBUF]

    zeros16 = jnp.zeros((16,), _F32)

    def _zero(i, _):
        for k in range(8):
            bufs[0][i, pl.ds(k * 16, 16)] = zeros16
        return 0

    lax.fori_loop(0, EB, _zero, 0)

    for h in range(NSL // EB):
        pltpu.async_copy(bufs[0], acc_sh.at[pl.ds(s * NSL + h * EB, EB)],
                         gsems[0])
    for h in range(NSL // EB):
        pltpu.make_async_copy(bufs[0], acc_sh.at[pl.ds(s * NSL + h * EB, EB)],
                              gsems[0]).wait()
    plsc.subcore_barrier()

    def _start_gather(j, b, iv):
        @pl.when(c == 0)
        def _():
            pltpu.async_copy(xp0_hbm.at[iv.at[j]], bufs[b], gsems[b])

        @pl.when(c == 1)
        def _():
            pltpu.async_copy(xp1_hbm.at[iv.at[j]], bufs[b], gsems[b])

    def _wait_gather(j, b, iv):
        @pl.when(c == 0)
        def _():
            pltpu.make_async_copy(xp0_hbm.at[iv.at[j]], bufs[b],
                                  gsems[b]).wait()

        @pl.when(c == 1)
        def _():
            pltpu.make_async_copy(xp1_hbm.at[iv.at[j]], bufs[b],
                                  gsems[b]).wait()

    def _start_scatter(j, b, iv):
        pltpu.async_copy(bufs[b], acc_sh.at[iv.at[j]], ssems[b], add=True)

    def _wait_scatter(j, b, iv):
        pltpu.make_async_copy(bufs[b], acc_sh.at[iv.at[j]], ssems[b]).wait()

    def _stage_idx(ck, sync):
        row0 = s * SPT + ck * CHUNK
        p = ck % 2
        if sync:
            pltpu.sync_copy(src_hbm.at[pl.ds(row0, CHUNK)], sidxs[p])
            pltpu.sync_copy(dst_hbm.at[pl.ds(row0, CHUNK)], didxs[p])
        else:
            pltpu.async_copy(src_hbm.at[pl.ds(row0, CHUNK)], sidxs[p],
                             isems[0])
            pltpu.async_copy(dst_hbm.at[pl.ds(row0, CHUNK)], didxs[p],
                             isems[1])

    def _wait_idx(ck):
        row0 = s * SPT + ck * CHUNK
        p = ck % 2
        pltpu.make_async_copy(src_hbm.at[pl.ds(row0, CHUNK)], sidxs[p],
                              isems[0]).wait()
        pltpu.make_async_copy(dst_hbm.at[pl.ds(row0, CHUNK)], didxs[p],
                              isems[1]).wait()

    NCK = SPT // CHUNK
    _stage_idx(0, True)
    for q in range(NBUF - 1):
        _start_gather(q, q, sidxs[0])

    for ck in range(NCK):
        si = sidxs[ck % 2]
        di = didxs[ck % 2]
        _wait_gather(0, 0, si)
        _start_scatter(0, 0, di)
        if ck > 0:
            _wait_scatter(CHUNK - 1, NBUF - 1, didxs[1 - ck % 2])
        if ck < NCK - 1:
            _stage_idx(ck + 1, False)
        _start_gather(NBUF - 1, NBUF - 1, si)

        @pl.loop(1, CHUNK - NBUF + 1, step=NBUF)
        def _edge(g):
            for db in range(NBUF):
                j = g + db
                b = (1 + db) % NBUF
                _wait_gather(j, b, si)
                _start_scatter(j, b, di)
                _wait_scatter(j - 1, db % NBUF, di)
                _start_gather(j + NBUF - 1, db % NBUF, si)

        for jj in range(CHUNK - NBUF + 1, CHUNK):
            _wait_gather(jj, jj % NBUF, si)
            _start_scatter(jj, jj % NBUF, di)
            _wait_scatter(jj - 1, (jj - 1) % NBUF, di)
        if ck < NCK - 1:
            _wait_idx(ck + 1)
            nsi = sidxs[(ck + 1) % 2]
            for q in range(NBUF - 1):
                _start_gather(q, q, nsi)
    _wait_scatter(CHUNK - 1, NBUF - 1, didxs[(NCK - 1) % 2])

    plsc.subcore_barrier()

    ND = NSL // EB

    def _drows(h):
        return pl.ds(s * NSL + h * EB, EB)

    def _start_out(h):
        @pl.when(c == 0)
        def _():
            pltpu.async_copy(bufs[h % 2], agg0_hbm.at[_drows(h)],
                             ssems[h % 2])

        @pl.when(c == 1)
        def _():
            pltpu.async_copy(bufs[h % 2], agg1_hbm.at[_drows(h)],
                             ssems[h % 2])

    def _wait_out(h):
        @pl.when(c == 0)
        def _():
            pltpu.make_async_copy(bufs[h % 2], agg0_hbm.at[_drows(h)],
                                  ssems[h % 2]).wait()

        @pl.when(c == 1)
        def _():
            pltpu.make_async_copy(bufs[h % 2], agg1_hbm.at[_drows(h)],
                                  ssems[h % 2]).wait()

    pltpu.async_copy(acc_sh.at[_drows(0)], bufs[0], gsems[0])
    for h in range(ND):
        pltpu.make_async_copy(acc_sh.at[_drows(h)], bufs[h % 2],
                              gsems[h % 2]).wait()
        _start_out(h)
        if h + 1 < ND:
            if h >= 1:
                _wait_out(h - 1)
            pltpu.async_copy(acc_sh.at[_drows(h + 1)], bufs[(h + 1) % 2],
                             gsems[(h + 1) % 2])
    _wait_out(ND - 2)
    _wait_out(ND - 1)


def _k3_body(agg0, agg1, xp0, xp1, dinv, w, b2, a2, u2, out_ref):
    w_ = w[...]
    u_ = u2[...]
    wv = jnp.dot(u_, w_, preferred_element_type=_F32)
    nv = jnp.sqrt(jnp.sum(wv * wv))
    v = wv / (nv + 1e-12)
    wv2 = lax.dot_general(v, w_, (((1,), (1,)), ((), ())),
                          preferred_element_type=_F32)
    nu = jnp.sqrt(jnp.sum(wv2 * wv2))
    sigma = jnp.sum(wv2 * wv2) / (nu + 1e-12)

    d = dinv[...]
    t0 = (agg0[...] + xp0[...]) * d
    t1 = (agg1[...] + xp1[...]) * d
    o = (jnp.dot(t0, w_[0:128, :], preferred_element_type=_F32)
         + jnp.dot(t1, w_[128:256, :], preferred_element_type=_F32))
    o = o * (1.0 / sigma) + b2[...]
    al = a2[0, 0]
    out_ref[...] = jnp.where(o >= 0, o, al * o)


@jax.jit
def kernel(x, edge_index, W, b, a, u):
    src = edge_index[0]
    dst = edge_index[1]
    pad = jnp.full((EP - E,), N, _I32)
    srcr = jnp.concatenate([src, pad]).reshape(ERW, EB)
    dstr = jnp.concatenate([dst, pad]).reshape(ERW, EB)
    x_pad = jnp.pad(x, ((0, NP - N), (0, 0)))

    mesh = plsc.VectorSubcoreMesh(core_axis_name="c", subcore_axis_name="s")

    k1 = pl.kernel(
        _k1_body,
        out_type=(
            jax.ShapeDtypeStruct((NP,), _F32),
            jax.ShapeDtypeStruct((NP, 128), _F32),
            jax.ShapeDtypeStruct((NP, 128), _F32),
        ),
        mesh=mesh,
        scratch_types=[
            pltpu.VMEM((SPT, EB), _I32),
            pltpu.VMEM((NP,), _F32),
            pltpu.VMEM((NSL,), _F32),
            pltpu.VMEM((16, NSL), _F32),
            pltpu.VMEM((NSL // 4, 128), _F32),
            pltpu.VMEM((NSL // 4, 128), _F32),
            pltpu.VMEM_SHARED((16, NP), _F32),
            pltpu.SemaphoreType.DMA,
            pltpu.SemaphoreType.DMA,
        ],
        compiler_params=pltpu.CompilerParams(needs_layout_passes=False),
    )
    dinv, xp0, xp1 = k1(x_pad, dstr)

    k2 = pl.kernel(
        _k2_body,
        out_type=(
            jax.ShapeDtypeStruct((NP, 128), _F32),
            jax.ShapeDtypeStruct((NP, 128), _F32),
        ),
        mesh=mesh,
        scratch_types=[pltpu.VMEM((CHUNK, EB), _I32)] * 4
        + [pltpu.VMEM((EB, 128), _F32)] * NBUF
        + [pltpu.VMEM_SHARED((NP, 128), _F32)]
        + [pltpu.SemaphoreType.DMA] * (2 * NBUF + 2),
        compiler_params=pltpu.CompilerParams(needs_layout_passes=False),
    )
    agg0, agg1 = k2(xp0, xp1, srcr, dstr)

    blk = 512
    grid = NP // blk
    outp = pl.pallas_call(
        _k3_body,
        grid=(grid,),
        in_specs=[
            pl.BlockSpec((blk, 128), lambda i: (i, 0)),
            pl.BlockSpec((blk, 128), lambda i: (i, 0)),
            pl.BlockSpec((blk, 128), lambda i: (i, 0)),
            pl.BlockSpec((blk, 128), lambda i: (i, 0)),
            pl.BlockSpec((blk, 1), lambda i: (i, 0)),
            pl.BlockSpec((NF, NH), lambda i: (0, 0)),
            pl.BlockSpec((1, NH), lambda i: (0, 0)),
            pl.BlockSpec((1, 1), lambda i: (0, 0)),
            pl.BlockSpec((1, NF), lambda i: (0, 0)),
        ],
        out_specs=pl.BlockSpec((blk, NH), lambda i: (i, 0)),
        out_shape=jax.ShapeDtypeStruct((N, NH), _F32),
    )(agg0, agg1, xp0, xp1, dinv.reshape(NP, 1), W,
      b.reshape(1, NH), a.reshape(1, 1), u.reshape(1, NF))

    return outp


# speedup vs baseline: 1.3289x; 1.0042x over previous
import functools

import jax
import jax.numpy as jnp
from jax import lax
from jax.experimental import pallas as pl
from jax.experimental.pallas import tpu as pltpu
from jax.experimental.pallas import tpu_sc as plsc

N = 10000
E = 160000
NF = 256
NH = 512

NP = 10240
EP = 163840
ER = EP // 128
RPT = ER // 16
NSL = NP // 16
EB = 64
NBUF = 4
ERW = EP // EB
SPT = ERW // 16
CHUNK = 32

_F32 = jnp.float32
_I32 = jnp.int32


def _fast_rsqrt(d):
    ih = plsc.bitcast(d, _I32)
    ih = jnp.int32(0x5F3759DF) - lax.shift_right_logical(ih, 1)
    y = plsc.bitcast(ih, _F32)
    for _ in range(3):
        y = y * (1.5 - 0.5 * d * y * y)
    return y


def _k1_body(x_hbm, dst_hbm, dinv_hbm, xp0_hbm, xp1_hbm,
             idx_v, deg_v, dv_v, tmp_v, xb_v, xb2_v, shd, xisem, xosem):
    c = lax.axis_index("c")
    s = lax.axis_index("s")
    base = s * NSL

    pltpu.sync_copy(dst_hbm.at[pl.ds(s * SPT, SPT)], idx_v)

    zeros16 = jnp.zeros((16,), _F32)

    def _zero(i, _):
        deg_v[pl.ds(pl.multiple_of(i * 16, 16), 16)] = zeros16
        return 0

    lax.fori_loop(0, NP // 16, _zero, 0)

    ones16 = jnp.ones((16,), _F32)

    def _scat(j, _):
        for k in range(EB // 16):
            iv = idx_v[j, pl.ds(k * 16, 16)]
            plsc.addupdate_scatter(deg_v, [iv], ones16)
        return 0

    lax.fori_loop(0, SPT, _scat, 0)

    pltpu.sync_copy(deg_v, shd.at[s])
    plsc.subcore_barrier()

    pltpu.sync_copy(shd.at[pl.ds(0, 16), pl.ds(base, NSL)], tmp_v)

    def _acc(i, _):
        sl = pl.ds(pl.multiple_of(i * 16, 16), 16)
        acc = tmp_v[0, sl]
        for t in range(1, 16):
            acc = acc + tmp_v[t, sl]
        dv_v[sl] = acc
        return 0

    lax.fori_loop(0, NSL // 16, _acc, 0)

    def _rs(i, _):
        sl = pl.ds(pl.multiple_of(i * 16, 16), 16)
        dv_v[sl] = _fast_rsqrt(dv_v[sl] + 1.0)
        return 0

    lax.fori_loop(0, NSL // 16, _rs, 0)

    @pl.when(c == 0)
    def _():
        pltpu.sync_copy(dv_v, dinv_hbm.at[pl.ds(base, NSL)])

    qr = NSL // 4
    xbs = (xb_v, xb2_v)

    def _xin(h):
        return pltpu.make_async_copy(
            x_hbm.at[pl.ds(base + h * qr, qr),
                     pl.ds(pl.multiple_of(c * 128, 128), 128)],
            xbs[h % 2], xisem)

    def _xout(h):
        def _mk0():
            return pltpu.make_async_copy(
                xbs[h % 2], xp0_hbm.at[pl.ds(base + h * qr, qr)], xosem)

        def _mk1():
            return pltpu.make_async_copy(
                xbs[h % 2], xp1_hbm.at[pl.ds(base + h * qr, qr)], xosem)

        return _mk0, _mk1

    _xin(0).start()
    _xin(1).start()
    for h in range(4):
        _xin(h).wait()

        def _scale(i, _):
            ridx = jnp.full((16,), h * qr + i, _I32)
            dsp = plsc.load_gather(dv_v, [ridx])
            for k in range(8):
                sl = pl.ds(k * 16, 16)
                xbs[h % 2][i, sl] = xbs[h % 2][i, sl] * dsp
            return 0

        lax.fori_loop(0, qr, _scale, 0)

        mk0, mk1 = _xout(h)

        @pl.when(c == 0)
        def _():
            mk0().start()

        @pl.when(c == 1)
        def _():
            mk1().start()

        if h + 2 < 4:
            @pl.when(c == 0)
            def _():
                mk0().wait()

            @pl.when(c == 1)
            def _():
                mk1().wait()

            _xin(h + 2).start()
    for h in range(2, 4):
        mk0, mk1 = _xout(h)

        @pl.when(c == 0)
        def _():
            mk0().wait()

        @pl.when(c == 1)
        def _():
            mk1().wait()


def _k2_body(xp0_hbm, xp1_hbm, src_hbm, dst_hbm, agg0_hbm, agg1_hbm, *scr):
    c = lax.axis_index("c")
    s = lax.axis_index("s")
    sidxs = scr[0:2]
    didxs = scr[2:4]
    bufs = scr[4:4 + NBUF]
    acc_sh = scr[4 + NBUF]
    gsems = scr[5 + NBUF:5 + 2 * NBUF]
    ssems = scr[5 + 2 * NBUF:5 + 3 * NBUF]
    isems = scr[5 + 3 * NBUF:7 + 3 * NBUF]

    def _start_gather(j, b, iv):
        @pl.when(c == 0)
        def _():
            pltpu.async_copy(xp0_hbm.at[iv.at[j]], bufs[b], gsems[b])

        @pl.when(c == 1)
        def _():
            pltpu.async_copy(xp1_hbm.at[iv.at[j]], bufs[b], gsems[b])

    def _wait_gather(j, b, iv):
        @pl.when(c == 0)
        def _():
            pltpu.make_async_copy(xp0_hbm.at[iv.at[j]], bufs[b],
                                  gsems[b]).wait()

        @pl.when(c == 1)
        def _():
            pltpu.make_async_copy(xp1_hbm.at[iv.at[j]], bufs[b],
                                  gsems[b]).wait()

    def _start_scatter(j, b, iv):
        pltpu.async_copy(bufs[b], acc_sh.at[iv.at[j]], ssems[b], add=True)

    def _wait_scatter(j, b, iv):
        pltpu.make_async_copy(bufs[b], acc_sh.at[iv.at[j]], ssems[b]).wait()

    def _stage_idx(ck, sync):
        row0 = s * SPT + ck * CHUNK
        p = ck % 2
        if sync:
            pltpu.sync_copy(src_hbm.at[pl.ds(row0, CHUNK)], sidxs[p])
            pltpu.sync_copy(dst_hbm.at[pl.ds(row0, CHUNK)], didxs[p])
        else:
            pltpu.async_copy(src_hbm.at[pl.ds(row0, CHUNK)], sidxs[p],
                             isems[0])
            pltpu.async_copy(dst_hbm.at[pl.ds(row0, CHUNK)], didxs[p],
                             isems[1])

    def _wait_idx(ck):
        row0 = s * SPT + ck * CHUNK
        p = ck % 2
        pltpu.make_async_copy(src_hbm.at[pl.ds(row0, CHUNK)], sidxs[p],
                              isems[0]).wait()
        pltpu.make_async_copy(dst_hbm.at[pl.ds(row0, CHUNK)], didxs[p],
                              isems[1]).wait()

    NCK = SPT // CHUNK
    _stage_idx(0, True)
    for q in range(NBUF - 1):
        _start_gather(q, q, sidxs[0])

    zeros16 = jnp.zeros((16,), _F32)

    def _zero(i, _):
        for k in range(8):
            bufs[NBUF - 1][i, pl.ds(k * 16, 16)] = zeros16
        return 0

    lax.fori_loop(0, EB, _zero, 0)

    for h in range(NSL // EB):
        pltpu.async_copy(bufs[NBUF - 1],
                         acc_sh.at[pl.ds(s * NSL + h * EB, EB)],
                         ssems[0])
    for h in range(NSL // EB):
        pltpu.make_async_copy(bufs[NBUF - 1],
                              acc_sh.at[pl.ds(s * NSL + h * EB, EB)],
                              ssems[0]).wait()
    plsc.subcore_barrier()

    for ck in range(NCK):
        si = sidxs[ck % 2]
        di = didxs[ck % 2]
        _wait_gather(0, 0, si)
        _start_scatter(0, 0, di)
        if ck > 0:
            _wait_scatter(CHUNK - 1, NBUF - 1, didxs[1 - ck % 2])
        if ck < NCK - 1:
            _stage_idx(ck + 1, False)
        _start_gather(NBUF - 1, NBUF - 1, si)

        @pl.loop(1, CHUNK - NBUF + 1, step=NBUF)
        def _edge(g):
            for db in range(NBUF):
                j = g + db
                b = (1 + db) % NBUF
                _wait_gather(j, b, si)
                _start_scatter(j, b, di)
                _wait_scatter(j - 1, db % NBUF, di)
                _start_gather(j + NBUF - 1, db % NBUF, si)

        for jj in range(CHUNK - NBUF + 1, CHUNK):
            _wait_gather(jj, jj % NBUF, si)
            _start_scatter(jj, jj % NBUF, di)
            _wait_scatter(jj - 1, (jj - 1) % NBUF, di)
        if ck < NCK - 1:
            _wait_idx(ck + 1)
            nsi = sidxs[(ck + 1) % 2]
            for q in range(NBUF - 1):
                _start_gather(q, q, nsi)
    _wait_scatter(CHUNK - 1, NBUF - 1, didxs[(NCK - 1) % 2])

    plsc.subcore_barrier()

    ND = NSL // EB

    def _drows(h):
        return pl.ds(s * NSL + h * EB, EB)

    def _start_out(h):
        @pl.when(c == 0)
        def _():
            pltpu.async_copy(bufs[h % 2], agg0_hbm.at[_drows(h)],
                             ssems[h % 2])

        @pl.when(c == 1)
        def _():
            pltpu.async_copy(bufs[h % 2], agg1_hbm.at[_drows(h)],
                             ssems[h % 2])

    def _wait_out(h):
        @pl.when(c == 0)
        def _():
            pltpu.make_async_copy(bufs[h % 2], agg0_hbm.at[_drows(h)],
                                  ssems[h % 2]).wait()

        @pl.when(c == 1)
        def _():
            pltpu.make_async_copy(bufs[h % 2], agg1_hbm.at[_drows(h)],
                                  ssems[h % 2]).wait()

    pltpu.async_copy(acc_sh.at[_drows(0)], bufs[0], gsems[0])
    for h in range(ND):
        pltpu.make_async_copy(acc_sh.at[_drows(h)], bufs[h % 2],
                              gsems[h % 2]).wait()
        _start_out(h)
        if h + 1 < ND:
            if h >= 1:
                _wait_out(h - 1)
            pltpu.async_copy(acc_sh.at[_drows(h + 1)], bufs[(h + 1) % 2],
                             gsems[(h + 1) % 2])
    _wait_out(ND - 2)
    _wait_out(ND - 1)


def _k3_body(agg0, agg1, xp0, xp1, dinv, w, b2, a2, u2, out_ref):
    w_ = w[...]
    u_ = u2[...]
    wv = jnp.dot(u_, w_, preferred_element_type=_F32)
    nv = jnp.sqrt(jnp.sum(wv * wv))
    v = wv / (nv + 1e-12)
    wv2 = lax.dot_general(v, w_, (((1,), (1,)), ((), ())),
                          preferred_element_type=_F32)
    nu = jnp.sqrt(jnp.sum(wv2 * wv2))
    sigma = jnp.sum(wv2 * wv2) / (nu + 1e-12)

    d = dinv[...]
    t0 = (agg0[...] + xp0[...]) * d
    t1 = (agg1[...] + xp1[...]) * d
    o = (jnp.dot(t0, w_[0:128, :], preferred_element_type=_F32)
         + jnp.dot(t1, w_[128:256, :], preferred_element_type=_F32))
    o = o * (1.0 / sigma) + b2[...]
    al = a2[0, 0]
    out_ref[...] = jnp.where(o >= 0, o, al * o)


@jax.jit
def kernel(x, edge_index, W, b, a, u):
    src = edge_index[0]
    dst = edge_index[1]
    pad = jnp.full((EP - E,), N, _I32)
    srcr = jnp.concatenate([src, pad]).reshape(ERW, EB)
    dstr = jnp.concatenate([dst, pad]).reshape(ERW, EB)
    x_pad = jnp.pad(x, ((0, NP - N), (0, 0)))

    mesh = plsc.VectorSubcoreMesh(core_axis_name="c", subcore_axis_name="s")

    k1 = pl.kernel(
        _k1_body,
        out_type=(
            jax.ShapeDtypeStruct((NP,), _F32),
            jax.ShapeDtypeStruct((NP, 128), _F32),
            jax.ShapeDtypeStruct((NP, 128), _F32),
        ),
        mesh=mesh,
        scratch_types=[
            pltpu.VMEM((SPT, EB), _I32),
            pltpu.VMEM((NP,), _F32),
            pltpu.VMEM((NSL,), _F32),
            pltpu.VMEM((16, NSL), _F32),
            pltpu.VMEM((NSL // 4, 128), _F32),
            pltpu.VMEM((NSL // 4, 128), _F32),
            pltpu.VMEM_SHARED((16, NP), _F32),
            pltpu.SemaphoreType.DMA,
            pltpu.SemaphoreType.DMA,
        ],
        compiler_params=pltpu.CompilerParams(needs_layout_passes=False),
    )
    dinv, xp0, xp1 = k1(x_pad, dstr)

    k2 = pl.kernel(
        _k2_body,
        out_type=(
            jax.ShapeDtypeStruct((NP, 128), _F32),
            jax.ShapeDtypeStruct((NP, 128), _F32),
        ),
        mesh=mesh,
        scratch_types=[pltpu.VMEM((CHUNK, EB), _I32)] * 4
        + [pltpu.VMEM((EB, 128), _F32)] * NBUF
        + [pltpu.VMEM_SHARED((NP, 128), _F32)]
        + [pltpu.SemaphoreType.DMA] * (2 * NBUF + 2),
        compiler_params=pltpu.CompilerParams(needs_layout_passes=False),
    )
    agg0, agg1 = k2(xp0, xp1, srcr, dstr)

    blk = 512
    grid = NP // blk
    outp = pl.pallas_call(
        _k3_body,
        grid=(grid,),
        in_specs=[
            pl.BlockSpec((blk, 128), lambda i: (i, 0)),
            pl.BlockSpec((blk, 128), lambda i: (i, 0)),
            pl.BlockSpec((blk, 128), lambda i: (i, 0)),
            pl.BlockSpec((blk, 128), lambda i: (i, 0)),
            pl.BlockSpec((blk, 1), lambda i: (i, 0)),
            pl.BlockSpec((NF, NH), lambda i: (0, 0)),
            pl.BlockSpec((1, NH), lambda i: (0, 0)),
            pl.BlockSpec((1, 1), lambda i: (0, 0)),
            pl.BlockSpec((1, NF), lambda i: (0, 0)),
        ],
        out_specs=pl.BlockSpec((blk, NH), lambda i: (i, 0)),
        out_shape=jax.ShapeDtypeStruct((N, NH), _F32),
    )(agg0, agg1, xp0, xp1, dinv.reshape(NP, 1), W,
      b.reshape(1, NH), a.reshape(1, 1), u.reshape(1, NF))

    return outp
